# baseline (speedup 1.0000x reference)
import jax
import jax.numpy as jnp
from jax import lax
from jax.experimental import pallas as pl
from jax.experimental.pallas import tpu as pltpu
from jax.experimental.pallas import tpu_sc as plsc

_BATCH = 1024
_SEQ = 200
_DIM = 64
_NC = 2
_NS = 16
_NW = _NC * _NS
_BPW = _BATCH // _NW
_NG = 13

_NBUF = 2


def _emb_body(idx_hbm, tok_hbm, pos_hbm, out_hbm, idx_v, pos_v, pair_v, wb_v,
              gsem, wsem):
    wid = lax.axis_index("s") * _NC + lax.axis_index("c")
    base_b = wid * _BPW

    pltpu.sync_copy(idx_hbm.at[pl.ds(base_b, _BPW)], idx_v)
    pltpu.sync_copy(pos_hbm, pos_v)

    def start_gather(r, buf):
        for g in range(_NG):
            o = min(g * 16, _SEQ - 16)
            ivec = idx_v[r, pl.ds(o, 16)]
            pltpu.async_copy(tok_hbm.at[lax.shift_right_logical(ivec, 1)],
                             pair_v.at[buf, pl.ds(o, 16)], gsem.at[buf])

    def wait_gather(buf):
        for g in range(_NG):
            o = min(g * 16, _SEQ - 16)
            pltpu.make_async_copy(tok_hbm.at[idx_v[0, pl.ds(0, 16)]],
                                  pair_v.at[buf, pl.ds(o, 16)],
                                  gsem.at[buf]).wait()

    def start_wb(r):
        pltpu.async_copy(wb_v, out_hbm.at[base_b + r], wsem)

    def wait_wb(r):
        pltpu.make_async_copy(wb_v, out_hbm.at[base_b + r], wsem).wait()

    start_gather(0, 0)

    def row_body(r, _):
        buf = lax.rem(r, _NBUF)
        nbuf = lax.rem(r + 1, _NBUF)

        @pl.when(r + 1 < _BPW)
        def _():
            start_gather(r + 1, nbuf)

        wait_gather(buf)

        @pl.when(r >= 1)
        def _():
            wait_wb(r - 1)

        def sel_add(l, par):
            for c in range(_DIM // 16):
                sl = pl.ds(c * 16, 16)
                lo = pair_v[buf, l, pl.ds(c * 16, 16)]
                hi = pair_v[buf, l, pl.ds(_DIM + c * 16, 16)]
                wb_v[l, sl] = lo + par * (hi - lo) + pos_v[l, sl]

        def add_group(g, _):
            o = g * 16
            fvec = (idx_v[r, pl.ds(o, 16)] & 1).astype(jnp.float32)
            for j in range(16):
                sel_add(o + j, jnp.broadcast_to(fvec[j], (16,)))
            return 0

        lax.fori_loop(0, _SEQ // 16, add_group, 0)
        tvec = (idx_v[r, pl.ds(_SEQ - 16, 16)] & 1).astype(jnp.float32)
        for j in range(8, 16):
            sel_add(_SEQ - 16 + j, jnp.broadcast_to(tvec[j], (16,)))
        start_wb(r)
        return 0

    lax.fori_loop(0, _BPW, row_body, 0)
    wait_wb(_BPW - 1)


@jax.jit
def _emb_call(idx, token_pairs, position_table):
    mesh = plsc.VectorSubcoreMesh(core_axis_name="c", subcore_axis_name="s")
    return pl.kernel(
        _emb_body,
        out_type=jax.ShapeDtypeStruct((_BATCH, _SEQ, _DIM), jnp.float32),
        mesh=mesh,
        scratch_types=[
            pltpu.VMEM((_BPW, _SEQ), jnp.int32),
            pltpu.VMEM((_SEQ, _DIM), jnp.float32),
            pltpu.VMEM((_NBUF, _SEQ, 2 * _DIM), jnp.float32),
            pltpu.VMEM((_SEQ, _DIM), jnp.float32),
            pltpu.SemaphoreType.DMA((_NBUF,)),
            pltpu.SemaphoreType.DMA,
        ],
    )(idx, token_pairs, position_table)


def kernel(inputs, token_table, position_table):
    token_pairs = token_table.reshape(500000, 128)
    return _emb_call(inputs.astype(jnp.int32), token_pairs, position_table)


# speedup vs baseline: 1.0941x; 1.0941x over previous
import jax
import jax.numpy as jnp
from jax import lax
from jax.experimental import pallas as pl
from jax.experimental.pallas import tpu as pltpu
from jax.experimental.pallas import tpu_sc as plsc

_BATCH = 1024
_SEQ = 200
_DIM = 64
_NC = 2
_NS = 16
_NW = _NC * _NS
_BPW = _BATCH // _NW

_NBUF = 8
_LOOKAHEAD = 5


def _emb_body(idx_hbm, tok_hbm, pos_hbm, out_hbm, idx_v, pos_v, rows_v, gsem, wsem):
    wid = lax.axis_index("s") * _NC + lax.axis_index("c")
    base_b = wid * _BPW

    pltpu.sync_copy(idx_hbm.at[pl.ds(base_b, _BPW)], idx_v)
    pltpu.sync_copy(pos_hbm, pos_v)

    def start_gather(r, buf):
        pltpu.async_copy(tok_hbm.at[idx_v.at[r]], rows_v.at[buf], gsem.at[buf])

    def wait_gather(buf):
        pltpu.make_async_copy(
            tok_hbm.at[idx_v.at[0]], rows_v.at[buf], gsem.at[buf]).wait()

    def start_wb(r, buf):
        pltpu.async_copy(rows_v.at[buf], out_hbm.at[base_b + r], wsem.at[buf])

    def wait_wb(buf):
        pltpu.make_async_copy(
            rows_v.at[buf], out_hbm.at[base_b], wsem.at[buf]).wait()

    for r in range(_LOOKAHEAD):
        start_gather(r, r % _NBUF)

    def chunk_body(r, _):
        buf = lax.rem(r, _NBUF)

        nxt = r + _LOOKAHEAD
        nbuf = lax.rem(nxt, _NBUF)

        @pl.when(r >= _NBUF - _LOOKAHEAD)
        def _():
            wait_wb(nbuf)

        @pl.when(nxt < _BPW)
        def _():
            start_gather(nxt, nbuf)

        wait_gather(buf)

        def add_body(l, _):
            for c in range(_DIM // 16):
                sl = pl.ds(c * 16, 16)
                rows_v[buf, l, sl] = rows_v[buf, l, sl] + pos_v[l, sl]
            return 0

        lax.fori_loop(0, _SEQ, add_body, 0, unroll=4)
        start_wb(r, buf)
        return 0

    lax.fori_loop(0, _BPW, chunk_body, 0)

    for r in range(_BPW - _NBUF + _LOOKAHEAD, _BPW):
        wait_wb(r % _NBUF)


@jax.jit
def _emb_call(idx, token_table, position_table):
    mesh = plsc.VectorSubcoreMesh(core_axis_name="c", subcore_axis_name="s")
    return pl.kernel(
        _emb_body,
        out_type=jax.ShapeDtypeStruct((_BATCH, _SEQ, _DIM), jnp.float32),
        mesh=mesh,
        scratch_types=[
            pltpu.VMEM((_BPW, _SEQ), jnp.int32),
            pltpu.VMEM((_SEQ, _DIM), jnp.float32),
            pltpu.VMEM((_NBUF, _SEQ, _DIM), jnp.float32),
            pltpu.SemaphoreType.DMA((_NBUF,)),
            pltpu.SemaphoreType.DMA((_NBUF,)),
        ],
        compiler_params=pltpu.CompilerParams(use_tc_tiling_on_sc=False),
    )(idx, token_table, position_table)


def kernel(inputs, token_table, position_table):
    return _emb_call(inputs.astype(jnp.int32), token_table, position_table)
